# baseline (speedup 1.0000x reference)
import functools

import jax
import jax.numpy as jnp
from jax.experimental import pallas as pl

_BLK_N = 2048
_EPS = 1e-12
_NEG = float("-inf")


def _topk_kernel(q_ref, m_ref, qn_ref, mn_ref, idx_ref, vals_ref, *,
                 n_valid, kk, blk_n, nsteps):
    i = pl.program_id(0)

    q = q_ref[...]
    qn = q / jnp.maximum(qn_ref[...], _EPS)
    m = m_ref[...]
    mn = m / jnp.maximum(mn_ref[...], _EPS)

    s = jax.lax.dot_general(
        qn.astype(jnp.bfloat16), mn.astype(jnp.bfloat16),
        (((1,), (1,)), ((), ())),
        preferred_element_type=jnp.float32)

    col0 = i * blk_n
    col = col0 + jax.lax.broadcasted_iota(jnp.int32, s.shape, 1)
    s = jnp.where(col < n_valid, s, _NEG)

    @pl.when(i == 0)
    def _():
        idx_ref[...] = jnp.zeros_like(idx_ref)
        vals_ref[...] = jnp.full_like(vals_ref, _NEG)

    rv = vals_ref[...]
    ri = idx_ref[...].astype(jnp.float32)
    slot = jax.lax.broadcasted_iota(jnp.int32, rv.shape, 1)

    def body(_, carry):
        def do(carry):
            s, rv, ri, _ = carry
            v = jnp.max(s, axis=1, keepdims=True)
            ci = jnp.argmax(s, axis=1).astype(jnp.int32)[:, None]
            cf = ci.astype(jnp.float32) + col0_f
            s2 = jnp.where(lcol == ci, _NEG, s)
            minv = jnp.min(rv, axis=1, keepdims=True)
            vic_cand = jnp.where(rv == minv, ri, -1.0)
            vic = jnp.argmax(vic_cand, axis=1).astype(jnp.int32)[:, None]
            vic_idx = jnp.max(vic_cand, axis=1, keepdims=True)
            ins = (v > minv) | ((v == minv) & (cf < vic_idx))
            hit = (slot == vic) & ins
            rv2 = jnp.where(hit, v, rv)
            ri2 = jnp.where(hit, cf, ri)
            alive = jnp.max(jnp.where(ins, 1.0, 0.0)) > 0.0
            return (s2, rv2, ri2, alive)
        return jax.lax.cond(carry[3], do, lambda x: x, carry)

    lcol = jax.lax.broadcasted_iota(jnp.int32, s.shape, 1)
    col0_f = col0.astype(jnp.float32)
    s, rv, ri, _ = jax.lax.fori_loop(
        0, kk, body, (s, rv, ri, jnp.bool_(True)))
    vals_ref[...] = rv
    idx_ref[...] = ri.astype(jnp.int32)

    @pl.when(i == nsteps - 1)
    def _():
        big = 1e9

        def emit(t, carry):
            rv, ri, sv, si = carry
            v = jnp.max(rv, axis=1, keepdims=True)
            c = jnp.min(jnp.where(rv == v, ri, big), axis=1, keepdims=True)
            taken = (rv == v) & (ri == c)
            rv2 = jnp.where(taken, _NEG, rv)
            ri2 = jnp.where(taken, big, ri)
            sv2 = jnp.where(slot == t, v, sv)
            si2 = jnp.where(slot == t, c, si)
            return (rv2, ri2, sv2, si2)

        rv2, ri2, sv, si = jax.lax.fori_loop(
            0, kk, emit, (rv, ri, jnp.zeros_like(rv), jnp.zeros_like(ri)))
        vals_ref[...] = sv
        idx_ref[...] = si.astype(jnp.int32)


def kernel(situation, memories, k):
    squeeze = situation.ndim == 1
    if squeeze:
        situation = situation[None, :]
    q_n, d = situation.shape
    n = memories.shape[0]
    kk = min(64, n)
    nsteps = (n + _BLK_N - 1) // _BLK_N

    kern = functools.partial(_topk_kernel, n_valid=n, kk=kk, blk_n=_BLK_N,
                             nsteps=nsteps)
    idx, vals = pl.pallas_call(
        kern,
        grid=(nsteps,),
        in_specs=[
            pl.BlockSpec((q_n, d), lambda i: (0, 0)),
            pl.BlockSpec((_BLK_N, d), lambda i: (i, 0)),
            pl.BlockSpec((q_n, 1), lambda i: (0, 0)),
            pl.BlockSpec((_BLK_N, 1), lambda i: (i, 0)),
        ],
        out_specs=[
            pl.BlockSpec((q_n, kk), lambda i: (0, 0)),
            pl.BlockSpec((q_n, kk), lambda i: (0, 0)),
        ],
        out_shape=[
            jax.ShapeDtypeStruct((q_n, kk), jnp.int32),
            jax.ShapeDtypeStruct((q_n, kk), jnp.float32),
        ],
    )(situation, memories,
      jnp.linalg.norm(situation, ord=2, axis=-1, keepdims=True),
      jnp.linalg.norm(memories, ord=2, axis=-1, keepdims=True))

    delta = (jnp.minimum(k, n) - kk).astype(idx.dtype)
    idx = idx + delta
    if squeeze:
        idx = jnp.squeeze(idx, axis=0)
        vals = jnp.squeeze(vals, axis=0)
    return (idx, vals)
